# Initial kernel scaffold; baseline (speedup 1.0000x reference)
import jax
import jax.numpy as jnp
from jax.experimental import pallas as pl


def kernel(x, edge_index, edge_weight, degree, W):
    raise NotImplementedError("write your pallas kernel here")


# speedup vs baseline: 5.0057x; 5.0057x over previous
import functools

import jax
import jax.numpy as jnp
from jax import lax
from jax.experimental import pallas as pl
from jax.experimental.pallas import tpu as pltpu
from jax.experimental.pallas import tpu_sc as plsc

N_NODES = 10000
N_EDGES = 320000
D_FEAT = 128
LAYERS = 3
ALPHA = 0.5

NC = 2
NS = 16
DH = D_FEAT // NC

CE = 128
CPT = 158
EPT = CE * CPT
E_PAD = EPT * NS
NBUF = 2

N_PAD = 10240
RPT = N_PAD // NS
RCH = 128

_sc_mesh = plsc.VectorSubcoreMesh(
    core_axis_name="c", subcore_axis_name="s", num_cores=NC, num_subcores=NS)


@functools.partial(
    pl.kernel,
    out_type=jax.ShapeDtypeStruct((NC, N_PAD, DH), jnp.float32),
    mesh=_sc_mesh,
    scratch_types=[
        pltpu.VMEM((CPT, CE), jnp.int32),
        pltpu.VMEM((CPT, CE), jnp.int32),
        pltpu.VMEM((CPT, CE), jnp.float32),
        pltpu.VMEM((CE, DH), jnp.float32),
        pltpu.VMEM((CE, DH), jnp.float32),
        pltpu.SemaphoreType.DMA,
        pltpu.SemaphoreType.DMA,
        pltpu.VMEM_SHARED((N_PAD, DH), jnp.float32),
    ],
    compiler_params=pltpu.CompilerParams(
        use_tc_tiling_on_sc=False, needs_layout_passes=False),
)
def _sc_agg(xs_hbm, src_hbm, dst_hbm, w_hbm, out_hbm,
            src_v, dst_v, w_v, b0, b1, s0, s1, acc_sh):
    c = lax.axis_index("c")
    s = lax.axis_index("s")
    bufs = (b0, b1)
    sems = (s0, s1)
    table = xs_hbm.at[c]

    def _zero_row(r, _):
        for f in range(DH // 16):
            b0[r, pl.ds(f * 16, 16)] = jnp.zeros((16,), jnp.float32)
        return 0
    lax.fori_loop(0, RCH, _zero_row, 0)
    for k in range(RPT // RCH):
        pltpu.sync_copy(b0, acc_sh.at[pl.ds(s * RPT + k * RCH, RCH)])
    plsc.subcore_barrier()

    pltpu.sync_copy(src_hbm.at[s], src_v)
    pltpu.sync_copy(dst_hbm.at[s], dst_v)
    pltpu.sync_copy(w_hbm.at[s], w_v)

    def _issue_gather(t, j):
        return pltpu.async_copy(table.at[src_v.at[t]], bufs[j], sems[j])

    def _process(t, j):
        pltpu.make_async_copy(table.at[src_v.at[t]], bufs[j], sems[j]).wait()
        buf = bufs[j]

        def _scale4(g, _):
            for jj in range(4):
                e = g * 4 + jj
                w_splat = plsc.load_gather(
                    w_v.at[t], [jnp.full((16,), e, jnp.int32)])
                for f in range(DH // 16):
                    buf[e, pl.ds(f * 16, 16)] = (
                        buf[e, pl.ds(f * 16, 16)] * w_splat)
            return 0
        lax.fori_loop(0, CE // 4, _scale4, 0)

        pltpu.sync_copy(buf, acc_sh.at[dst_v.at[t]], add=True)

    for j in range(NBUF):
        _issue_gather(j, j)

    def _body(v, _):
        t = v * NBUF
        for j in range(NBUF):
            _process(t + j, j)

            @pl.when(t + j + NBUF < CPT)
            def _():
                _issue_gather(t + j + NBUF, j)
        return 0
    lax.fori_loop(0, CPT // NBUF, _body, 0)
    for j in range(CPT - NBUF * (CPT // NBUF)):
        _process(NBUF * (CPT // NBUF) + j, j)

    plsc.subcore_barrier()
    for k in range(RPT // RCH):
        base = s * RPT + k * RCH
        pltpu.sync_copy(acc_sh.at[pl.ds(base, RCH)], b0)
        pltpu.sync_copy(b0, out_hbm.at[c].at[pl.ds(base, RCH)])


_RB = 1000


def _prescale_body(x_ref, w_ref, xs_ref):
    for h in range(NC):
        xs_ref[h] = (x_ref[:, h * DH:(h + 1) * DH]
                     * w_ref[:, h * DH:(h + 1) * DH])


def _elem_body(x_ref, agg_ref, deg_ref, w_ref, xn_ref, xsn_ref):
    d = deg_ref[...]
    scale = (1.0 - ALPHA) * d * d
    h0 = ALPHA * x_ref[:, :DH] + scale * agg_ref[0]
    h1 = ALPHA * x_ref[:, DH:] + scale * agg_ref[1]
    inv = 1.0 / (jnp.sum(h0, axis=1, keepdims=True)
                 + jnp.sum(h1, axis=1, keepdims=True))
    h0 = h0 * inv
    h1 = h1 * inv
    xn_ref[:, :DH] = h0
    xn_ref[:, DH:] = h1
    xsn_ref[0] = h0 * w_ref[:, :DH]
    xsn_ref[1] = h1 * w_ref[:, DH:]


def _prescale(x, W):
    return pl.pallas_call(
        _prescale_body,
        grid=(N_NODES // _RB,),
        in_specs=[
            pl.BlockSpec((_RB, D_FEAT), lambda i: (i, 0)),
            pl.BlockSpec((1, D_FEAT), lambda i: (0, 0)),
        ],
        out_specs=pl.BlockSpec((NC, _RB, DH), lambda i: (0, i, 0)),
        out_shape=jax.ShapeDtypeStruct((NC, N_NODES, DH), jnp.float32),
    )(x, W)


def _elemwise(x, agg, deg_col, W):
    return pl.pallas_call(
        _elem_body,
        grid=(N_NODES // _RB,),
        in_specs=[
            pl.BlockSpec((_RB, D_FEAT), lambda i: (i, 0)),
            pl.BlockSpec((NC, _RB, DH), lambda i: (0, i, 0)),
            pl.BlockSpec((_RB, 1), lambda i: (i, 0)),
            pl.BlockSpec((1, D_FEAT), lambda i: (0, 0)),
        ],
        out_specs=[
            pl.BlockSpec((_RB, D_FEAT), lambda i: (i, 0)),
            pl.BlockSpec((NC, _RB, DH), lambda i: (0, i, 0)),
        ],
        out_shape=[
            jax.ShapeDtypeStruct((N_NODES, D_FEAT), jnp.float32),
            jax.ShapeDtypeStruct((NC, N_NODES, DH), jnp.float32),
        ],
    )(x, agg, deg_col, W)


@jax.jit
def kernel(x, edge_index, edge_weight, degree, W):
    src = edge_index[0].astype(jnp.int32)
    dst = edge_index[1].astype(jnp.int32)
    pad = E_PAD - N_EDGES
    src3 = jnp.pad(src, (0, pad)).reshape(NS, CPT, CE)
    dst3 = jnp.pad(dst, (0, pad)).reshape(NS, CPT, CE)
    w3 = jnp.pad(edge_weight, (0, pad)).reshape(NS, CPT, CE)
    deg_col = degree[:, None]

    xs = _prescale(x, W)
    for _ in range(LAYERS):
        agg = _sc_agg(xs, src3, dst3, w3)
        x, xs = _elemwise(x, agg, deg_col, W)
    return x


# speedup vs baseline: 6.3068x; 1.2599x over previous
import functools

import jax
import jax.numpy as jnp
from jax import lax
from jax.experimental import pallas as pl
from jax.experimental.pallas import tpu as pltpu
from jax.experimental.pallas import tpu_sc as plsc

N_NODES = 10000
N_EDGES = 320000
D_FEAT = 128
LAYERS = 3
ALPHA = 0.5

NC = 2
NS = 16
DH = D_FEAT // NC

CE = 128
CPT = 158
EPT = CE * CPT
E_PAD = EPT * NS
NBUF = 3

N_PAD = 10240
RPT = N_PAD // NS
RCH = 128

_sc_mesh = plsc.VectorSubcoreMesh(
    core_axis_name="c", subcore_axis_name="s", num_cores=NC, num_subcores=NS)


@functools.partial(
    pl.kernel,
    out_type=jax.ShapeDtypeStruct((NC, N_PAD, DH), jnp.float32),
    mesh=_sc_mesh,
    scratch_types=[
        pltpu.VMEM((CPT, CE), jnp.int32),
        pltpu.VMEM((CPT, CE), jnp.int32),
        pltpu.VMEM((CPT, CE), jnp.float32),
        pltpu.VMEM((CE, DH), jnp.float32),
        pltpu.VMEM((CE, DH), jnp.float32),
        pltpu.VMEM((CE, DH), jnp.float32),
        pltpu.SemaphoreType.DMA,
        pltpu.SemaphoreType.DMA,
        pltpu.SemaphoreType.DMA,
        pltpu.SemaphoreType.DMA,
        pltpu.SemaphoreType.DMA,
        pltpu.SemaphoreType.DMA,
        pltpu.VMEM_SHARED((N_PAD, DH), jnp.float32),
    ],
    compiler_params=pltpu.CompilerParams(
        use_tc_tiling_on_sc=False, needs_layout_passes=False),
)
def _sc_agg(xs_hbm, src_hbm, dst_hbm, w_hbm, out_hbm,
            src_v, dst_v, w_v, b0, b1, b2, g0, g1, g2, u0, u1, u2, acc_sh):
    c = lax.axis_index("c")
    s = lax.axis_index("s")
    bufs = (b0, b1, b2)
    gsems = (g0, g1, g2)
    usems = (u0, u1, u2)
    table = xs_hbm.at[c]

    def _zero_row(r, _):
        for f in range(DH // 16):
            b0[r, pl.ds(f * 16, 16)] = jnp.zeros((16,), jnp.float32)
        return 0
    lax.fori_loop(0, RCH, _zero_row, 0)
    for k in range(RPT // RCH):
        pltpu.sync_copy(b0, acc_sh.at[pl.ds(s * RPT + k * RCH, RCH)])
    plsc.subcore_barrier()

    pltpu.sync_copy(src_hbm.at[s], src_v)
    pltpu.sync_copy(dst_hbm.at[s], dst_v)
    pltpu.sync_copy(w_hbm.at[s], w_v)

    def _issue_gather(t, j):
        pltpu.async_copy(table.at[src_v.at[t]], bufs[j], gsems[j])

    def _wait_gather(t, j):
        pltpu.make_async_copy(table.at[src_v.at[t]], bufs[j], gsems[j]).wait()

    def _issue_scatter(t, j):
        pltpu.async_copy(bufs[j], acc_sh.at[dst_v.at[t]], usems[j], add=True)

    def _wait_scatter(t, j):
        pltpu.make_async_copy(bufs[j], acc_sh.at[dst_v.at[t]], usems[j]).wait()

    def _scale(t, j):
        buf = bufs[j]

        @plsc.parallel_loop(0, CE, 4, unroll=2)
        def _scale4(e0):
            for jj in range(4):
                e = e0 + jj
                w_splat = plsc.load_gather(
                    w_v.at[t], [jnp.full((16,), e, jnp.int32)])
                for f in range(DH // 16):
                    buf[e, pl.ds(f * 16, 16)] = (
                        buf[e, pl.ds(f * 16, 16)] * w_splat)

    _issue_gather(0, 0)
    _issue_gather(1, 1)

    def _chunk_steady(t, j, jprev):
        _wait_gather(t, j)
        _scale(t, j)
        _issue_scatter(t, j)
        _wait_scatter(t - 1, jprev)
        _issue_gather(t + 2, jprev)

    _wait_gather(0, 0)
    _scale(0, 0)
    _issue_scatter(0, 0)
    _issue_gather(2, 2)
    _chunk_steady(1, 1, 0)
    _chunk_steady(2, 2, 1)

    def _body(v, _):
        t = v * 3
        _chunk_steady(t, 0, 2)
        _chunk_steady(t + 1, 1, 0)
        _chunk_steady(t + 2, 2, 1)
        return 0
    lax.fori_loop(1, CPT // 3, _body, 0)

    _wait_gather(156, 0)
    _scale(156, 0)
    _issue_scatter(156, 0)
    _wait_scatter(155, 2)
    _chunk_tail = None
    _wait_gather(157, 1)
    _scale(157, 1)
    _issue_scatter(157, 1)
    _wait_scatter(156, 0)
    _wait_scatter(157, 1)

    plsc.subcore_barrier()
    for k in range(RPT // RCH):
        base = s * RPT + k * RCH
        pltpu.sync_copy(acc_sh.at[pl.ds(base, RCH)],
                        out_hbm.at[c].at[pl.ds(base, RCH)])


_RB = 1000


def _prescale_body(x_ref, w_ref, xs_ref):
    for h in range(NC):
        xs_ref[h] = (x_ref[:, h * DH:(h + 1) * DH]
                     * w_ref[:, h * DH:(h + 1) * DH])


def _elem_body(x_ref, agg_ref, deg_ref, w_ref, xn_ref, xsn_ref):
    d = deg_ref[...]
    scale = (1.0 - ALPHA) * d * d
    h0 = ALPHA * x_ref[:, :DH] + scale * agg_ref[0]
    h1 = ALPHA * x_ref[:, DH:] + scale * agg_ref[1]
    inv = 1.0 / (jnp.sum(h0, axis=1, keepdims=True)
                 + jnp.sum(h1, axis=1, keepdims=True))
    h0 = h0 * inv
    h1 = h1 * inv
    xn_ref[:, :DH] = h0
    xn_ref[:, DH:] = h1
    xsn_ref[0] = h0 * w_ref[:, :DH]
    xsn_ref[1] = h1 * w_ref[:, DH:]


def _prescale(x, W):
    return pl.pallas_call(
        _prescale_body,
        grid=(N_NODES // _RB,),
        in_specs=[
            pl.BlockSpec((_RB, D_FEAT), lambda i: (i, 0)),
            pl.BlockSpec((1, D_FEAT), lambda i: (0, 0)),
        ],
        out_specs=pl.BlockSpec((NC, _RB, DH), lambda i: (0, i, 0)),
        out_shape=jax.ShapeDtypeStruct((NC, N_NODES, DH), jnp.float32),
    )(x, W)


def _elemwise(x, agg, deg_col, W):
    return pl.pallas_call(
        _elem_body,
        grid=(N_NODES // _RB,),
        in_specs=[
            pl.BlockSpec((_RB, D_FEAT), lambda i: (i, 0)),
            pl.BlockSpec((NC, _RB, DH), lambda i: (0, i, 0)),
            pl.BlockSpec((_RB, 1), lambda i: (i, 0)),
            pl.BlockSpec((1, D_FEAT), lambda i: (0, 0)),
        ],
        out_specs=[
            pl.BlockSpec((_RB, D_FEAT), lambda i: (i, 0)),
            pl.BlockSpec((NC, _RB, DH), lambda i: (0, i, 0)),
        ],
        out_shape=[
            jax.ShapeDtypeStruct((N_NODES, D_FEAT), jnp.float32),
            jax.ShapeDtypeStruct((NC, N_NODES, DH), jnp.float32),
        ],
    )(x, agg, deg_col, W)


@jax.jit
def kernel(x, edge_index, edge_weight, degree, W):
    src = edge_index[0].astype(jnp.int32)
    dst = edge_index[1].astype(jnp.int32)
    pad = E_PAD - N_EDGES
    src3 = jnp.pad(src, (0, pad)).reshape(NS, CPT, CE)
    dst3 = jnp.pad(dst, (0, pad)).reshape(NS, CPT, CE)
    w3 = jnp.pad(edge_weight, (0, pad)).reshape(NS, CPT, CE)
    deg_col = degree[:, None]

    xs = _prescale(x, W)
    for _ in range(LAYERS):
        agg = _sc_agg(xs, src3, dst3, w3)
        x, xs = _elemwise(x, agg, deg_col, W)
    return x
